# Initial kernel scaffold; baseline (speedup 1.0000x reference)
import jax
import jax.numpy as jnp
from jax.experimental import pallas as pl


def kernel(input_ids, embed_table, role_table, gate_W, gate_b):
    raise NotImplementedError("write your pallas kernel here")


# speedup vs baseline: 46.5496x; 46.5496x over previous
import functools

import jax
import jax.numpy as jnp
from jax import lax
from jax.experimental import pallas as pl
from jax.experimental.pallas import tpu as pltpu
from jax.experimental.pallas import tpu_sc as plsc

_OFF = 3
_DOT = ord('.') + _OFF
_OB = ord('[') + _OFF
_CB = ord(']') + _OFF
_X = ord('x') + _OFF
_GAP = tuple(ord(c) + _OFF for c in '<gap>')
_NEG = -(1 << 30)
_PAD = 128

ROLE_PAD, ROLE_ALPHA, ROLE_PUNCT, ROLE_BOUND, ROLE_DAMAGE, ROLE_NUM = 0, 1, 2, 3, 4, 5


def _prep_kernel(B, S, V, D, R,
                 tids_ref, e_ref, r_ref, wt_ref, b_ref,
                 tbl_ref, idx_ref, cm_ref, sh_ref):
    t = tids_ref[...]
    i32 = jnp.int32

    def shifted(x, fill, ks):
        sh_ref[:, 0:_PAD] = jnp.full((B, _PAD), fill, i32)
        sh_ref[:, _PAD:_PAD + S] = x
        sh_ref[:, _PAD + S:] = jnp.full((B, _PAD), fill, i32)
        return [sh_ref[:, _PAD + k:_PAD + k + S] for k in ks]

    def cummax(x):
        cm_ref[:, 0:S] = jnp.full((B, S), _NEG, i32)
        cur = x
        sh = 1
        while sh < S:
            cm_ref[:, S:] = cur
            cur = jnp.maximum(cur, cm_ref[:, S - sh:2 * S - sh])
            sh *= 2
        return cur

    pad = (t < 3) | (t >= 259)
    punct = ((t == ord('-') + _OFF) | (t == ord('.') + _OFF)
             | (t == ord(':') + _OFF) | (t == ord('=') + _OFF)
             | (t == ord(' ') + _OFF))
    bound = ((t == ord('{') + _OFF) | (t == ord('}') + _OFF)
             | (t == ord('(') + _OFF) | (t == ord(')') + _OFF)
             | (t == _OB) | (t == _CB))
    num = (t >= ord('0') + _OFF) & (t <= ord('9') + _OFF)
    base = jnp.where(pad, ROLE_PAD,
                     jnp.where(punct, ROLE_PUNCT,
                               jnp.where(bound, ROLE_BOUND,
                                         jnp.where(num, ROLE_NUM, ROLE_ALPHA))))

    iota = lax.broadcasted_iota(i32, (B, S), 1)

    tf = shifted(t, -1, list(range(0, 5)))
    m5 = (tf[0] == _GAP[0])
    for k in range(1, 5):
        m5 = m5 & (tf[k] == _GAP[k])
    mb = shifted(m5.astype(i32), 0, list(range(-4, 1)))
    dgap = (mb[4] | mb[3] | mb[2] | mb[1] | mb[0]) != 0

    is_dot = t == _DOT
    df = shifted(is_dot.astype(i32), 0, [1, 2])
    m3 = is_dot & (df[0] != 0) & (df[1] != 0)
    lnd = cummax(jnp.where(is_dot, -1, iota))
    sel3 = m3 & ((iota - lnd - 1) % 3 == 0)
    sb = shifted(sel3.astype(i32), 0, [-2, -1, 0])
    ddot = (sb[0] | sb[1] | sb[2]) != 0

    enc = jnp.where(t == _OB, 2 * iota + 1,
                    jnp.where(t == _CB, 2 * iota, -1))
    c = cummax(enc)
    markx = (t == _X) & (c >= 0) & ((c & 1) == 1)

    damage = dgap | ddot | markx
    role = jnp.where(damage, ROLE_DAMAGE, base)
    idx_ref[...] = role * V + t

    e = e_ref[...]
    p = lax.dot_general(e, wt_ref[0:D, :], (((1,), (0,)), ((), ())),
                        preferred_element_type=jnp.float32)
    q = lax.dot_general(r_ref[...], wt_ref[D:2 * D, :], (((1,), (0,)), ((), ())),
                        preferred_element_type=jnp.float32)
    bvec = b_ref[...]
    for r in range(R):
        gate = jax.nn.sigmoid(p + q[r:r + 1, :] + bvec)
        tbl_ref[r * V:(r + 1) * V, :] = e + gate * r_ref[r:r + 1, :]


def _build_prep(B, S, V, D, R):
    return pl.pallas_call(
        functools.partial(_prep_kernel, B, S, V, D, R),
        out_shape=(
            jax.ShapeDtypeStruct((R * V, D), jnp.float32),
            jax.ShapeDtypeStruct((B, S), jnp.int32),
        ),
        scratch_shapes=[
            pltpu.VMEM((B, 2 * S), jnp.int32),
            pltpu.VMEM((B, S + 2 * _PAD), jnp.int32),
        ],
    )


def _build_gather(N, D, table_rows):
    NC, NS = 2, 16
    NW = NC * NS
    assert N % NW == 0
    b_per_w = N // NW
    chunk = 64
    assert b_per_w % chunk == 0
    n_chunks = b_per_w // chunk
    mesh = plsc.VectorSubcoreMesh(core_axis_name="c", subcore_axis_name="s")

    @functools.partial(
        pl.kernel,
        out_type=jax.ShapeDtypeStruct((N, D), jnp.float32),
        mesh=mesh,
        scratch_types=[
            pltpu.VMEM((chunk,), jnp.int32),
            pltpu.VMEM((chunk, D), jnp.float32),
            pltpu.SemaphoreType.DMA,
        ],
    )
    def gather(tbl_hbm, idx_hbm, out_hbm, idx_v, rows_v, sem):
        wid = lax.axis_index("s") * NC + lax.axis_index("c")
        base = wid * b_per_w
        for ci in range(n_chunks):
            off = base + ci * chunk
            pltpu.sync_copy(idx_hbm.at[pl.ds(off, chunk)], idx_v)
            pltpu.async_copy(tbl_hbm.at[idx_v], rows_v, sem).wait()
            pltpu.sync_copy(rows_v, out_hbm.at[pl.ds(off, chunk)])

    return gather


def kernel(input_ids, embed_table, role_table, gate_W, gate_b):
    B, S = input_ids.shape
    V, D = embed_table.shape
    R = role_table.shape[0]
    tids = input_ids.astype(jnp.int32)
    wt = gate_W.T
    b2 = gate_b.reshape(1, D)
    tbl, fidx = _build_prep(B, S, V, D, R)(tids, embed_table, role_table, wt, b2)
    out = _build_gather(B * S, D, R * V)(tbl, fidx.reshape(B * S))
    return out.reshape(B, S, D)


# speedup vs baseline: 50.3192x; 1.0810x over previous
import functools

import jax
import jax.numpy as jnp
from jax import lax
from jax.experimental import pallas as pl
from jax.experimental.pallas import tpu as pltpu
from jax.experimental.pallas import tpu_sc as plsc

_OFF = 3
_DOT = ord('.') + _OFF
_OB = ord('[') + _OFF
_CB = ord(']') + _OFF
_X = ord('x') + _OFF
_GAP = tuple(ord(c) + _OFF for c in '<gap>')
_NEG = -(1 << 30)
_PAD = 128

ROLE_PAD, ROLE_ALPHA, ROLE_PUNCT, ROLE_BOUND, ROLE_DAMAGE, ROLE_NUM = 0, 1, 2, 3, 4, 5


def _prep_kernel(B, S, V, D, R,
                 tids_ref, e_ref, r_ref, w_ref, b_ref,
                 tbl_ref, idx_ref, cm_ref, sh_ref, tv_ref, dsem):
    i32 = jnp.int32

    e = e_ref[...]
    p = lax.dot_general(e.astype(jnp.bfloat16), w_ref[:, 0:D],
                        (((1,), (1,)), ((), ())),
                        preferred_element_type=jnp.float32)
    q = lax.dot_general(r_ref[...].astype(jnp.bfloat16), w_ref[:, D:2 * D],
                        (((1,), (1,)), ((), ())),
                        preferred_element_type=jnp.float32)
    bvec = b_ref[...]
    for r in range(R):
        gate = jax.nn.sigmoid(p + q[r:r + 1, :] + bvec)
        tv_ref[r * V:(r + 1) * V, :] = e + gate * r_ref[r:r + 1, :]
    tbl_dma = pltpu.make_async_copy(tv_ref, tbl_ref, dsem)
    tbl_dma.start()

    t = tids_ref[...]

    def shifted(x, fill, ks):
        sh_ref[:, 0:_PAD] = jnp.full((B, _PAD), fill, i32)
        sh_ref[:, _PAD:_PAD + S] = x
        sh_ref[:, _PAD + S:] = jnp.full((B, _PAD), fill, i32)
        return [sh_ref[:, _PAD + k:_PAD + k + S] for k in ks]

    def cummax(x):
        cm_ref[:, 0:S] = jnp.full((B, S), _NEG, i32)
        cur = x
        sh = 1
        while sh < S:
            cm_ref[:, S:] = cur
            cur = jnp.maximum(cur, cm_ref[:, S - sh:2 * S - sh])
            sh *= 2
        return cur

    pad = (t < 3) | (t >= 259)
    punct = ((t == ord('-') + _OFF) | (t == ord('.') + _OFF)
             | (t == ord(':') + _OFF) | (t == ord('=') + _OFF)
             | (t == ord(' ') + _OFF))
    bound = ((t == ord('{') + _OFF) | (t == ord('}') + _OFF)
             | (t == ord('(') + _OFF) | (t == ord(')') + _OFF)
             | (t == _OB) | (t == _CB))
    num = (t >= ord('0') + _OFF) & (t <= ord('9') + _OFF)
    base = jnp.where(pad, ROLE_PAD,
                     jnp.where(punct, ROLE_PUNCT,
                               jnp.where(bound, ROLE_BOUND,
                                         jnp.where(num, ROLE_NUM, ROLE_ALPHA))))

    iota = lax.broadcasted_iota(i32, (B, S), 1)

    tf = shifted(t, -1, list(range(0, 5)))
    m5 = (tf[0] == _GAP[0])
    for k in range(1, 5):
        m5 = m5 & (tf[k] == _GAP[k])
    mb = shifted(m5.astype(i32), 0, list(range(-4, 1)))
    dgap = (mb[4] | mb[3] | mb[2] | mb[1] | mb[0]) != 0

    is_dot = t == _DOT
    df = shifted(is_dot.astype(i32), 0, [1, 2])
    m3 = is_dot & (df[0] != 0) & (df[1] != 0)
    lnd = cummax(jnp.where(is_dot, -1, iota))
    sel3 = m3 & ((iota - lnd - 1) % 3 == 0)
    sb = shifted(sel3.astype(i32), 0, [-2, -1, 0])
    ddot = (sb[0] | sb[1] | sb[2]) != 0

    enc = jnp.where(t == _OB, 2 * iota + 1,
                    jnp.where(t == _CB, 2 * iota, -1))
    c = cummax(enc)
    markx = (t == _X) & (c >= 0) & ((c & 1) == 1)

    damage = dgap | ddot | markx
    role = jnp.where(damage, ROLE_DAMAGE, base)
    idx_ref[...] = role * V + t
    tbl_dma.wait()


def _build_prep(B, S, V, D, R):
    return pl.pallas_call(
        functools.partial(_prep_kernel, B, S, V, D, R),
        out_shape=(
            jax.ShapeDtypeStruct((R * V, D), jnp.float32),
            jax.ShapeDtypeStruct((B, S), jnp.int32),
        ),
        out_specs=(
            pl.BlockSpec(memory_space=pl.ANY),
            pl.BlockSpec(memory_space=pltpu.MemorySpace.VMEM),
        ),
        scratch_shapes=[
            pltpu.VMEM((B, 2 * S), jnp.int32),
            pltpu.VMEM((B, S + 2 * _PAD), jnp.int32),
            pltpu.VMEM((R * V, D), jnp.float32),
            pltpu.SemaphoreType.DMA,
        ],
    )


def _build_gather(N, D, table_rows):
    NC, NS = 2, 16
    NW = NC * NS
    assert N % NW == 0
    b_per_w = N // NW
    chunk = 16
    nbuf = 6
    assert b_per_w % chunk == 0
    n_chunks = b_per_w // chunk
    mesh = plsc.VectorSubcoreMesh(core_axis_name="c", subcore_axis_name="s")

    scratch = ([pltpu.VMEM((b_per_w,), jnp.int32)]
               + [pltpu.VMEM((chunk, D), jnp.float32)] * nbuf
               + [pltpu.SemaphoreType.DMA] * (2 * nbuf))

    @functools.partial(
        pl.kernel,
        out_type=jax.ShapeDtypeStruct((N, D), jnp.float32),
        mesh=mesh,
        scratch_types=scratch,
    )
    def gather(tbl_hbm, idx_hbm, out_hbm, idx_all, *bufs):
        rowb = bufs[0:nbuf]
        gsem = bufs[nbuf:2 * nbuf]
        ssem = bufs[2 * nbuf:3 * nbuf]
        wid = lax.axis_index("s") * NC + lax.axis_index("c")
        base = wid * b_per_w
        pltpu.sync_copy(idx_hbm.at[pl.ds(base, b_per_w)], idx_all)
        pend_g = [None] * nbuf
        pend_s = [None] * nbuf
        ahead = nbuf - 1
        for ci in range(min(ahead, n_chunks)):
            pend_g[ci % nbuf] = pltpu.async_copy(
                tbl_hbm.at[idx_all.at[pl.ds(ci * chunk, chunk)]],
                rowb[ci % nbuf], gsem[ci % nbuf])
        for ci in range(n_chunks):
            b = ci % nbuf
            pend_g[b].wait()
            pend_s[b] = pltpu.async_copy(rowb[b],
                                         out_hbm.at[pl.ds(base + ci * chunk, chunk)],
                                         ssem[b])
            nxt = ci + ahead
            if nxt < n_chunks:
                nb = nxt % nbuf
                if pend_s[nb] is not None:
                    pend_s[nb].wait()
                    pend_s[nb] = None
                pend_g[nb] = pltpu.async_copy(
                    tbl_hbm.at[idx_all.at[pl.ds(nxt * chunk, chunk)]],
                    rowb[nb], gsem[nb])
        for p in pend_s:
            if p is not None:
                p.wait()

    return gather


def kernel(input_ids, embed_table, role_table, gate_W, gate_b):
    B, S = input_ids.shape
    V, D = embed_table.shape
    R = role_table.shape[0]
    tids = input_ids.astype(jnp.int32)
    b2 = gate_b.reshape(1, D)
    gw = gate_W.astype(jnp.bfloat16)
    tbl, fidx = _build_prep(B, S, V, D, R)(tids, embed_table, role_table, gw, b2)
    out = _build_gather(B * S, D, R * V)(tbl, fidx.reshape(B * S))
    return out.reshape(B, S, D)


# speedup vs baseline: 54.8054x; 1.0892x over previous
import functools

import jax
import jax.numpy as jnp
from jax import lax
from jax.experimental import pallas as pl
from jax.experimental.pallas import tpu as pltpu
from jax.experimental.pallas import tpu_sc as plsc

_OFF = 3
_DOT = ord('.') + _OFF
_OB = ord('[') + _OFF
_CB = ord(']') + _OFF
_X = ord('x') + _OFF
_GAP = tuple(ord(c) + _OFF for c in '<gap>')
_NEG = -(1 << 30)
_PAD = 128

ROLE_PAD, ROLE_ALPHA, ROLE_PUNCT, ROLE_BOUND, ROLE_DAMAGE, ROLE_NUM = 0, 1, 2, 3, 4, 5


def _prep_kernel(B, S, V, D, R,
                 tids_ref, e_ref, r_ref, w_ref, b_ref,
                 tbl_ref, idx_ref, cm_ref, sh_ref, tv_ref, dsem):
    i32 = jnp.int32

    e = e_ref[...]
    p = lax.dot_general(e, w_ref[:, 0:D], (((1,), (1,)), ((), ())),
                        preferred_element_type=jnp.float32)
    q = lax.dot_general(r_ref[...], w_ref[:, D:2 * D], (((1,), (1,)), ((), ())),
                        preferred_element_type=jnp.float32)
    bvec = b_ref[...]
    for r in range(R):
        gate = jax.nn.sigmoid(p + q[r:r + 1, :] + bvec)
        tv_ref[r * V:(r + 1) * V, :] = e + gate * r_ref[r:r + 1, :]
    tbl_dma = pltpu.make_async_copy(tv_ref, tbl_ref, dsem)
    tbl_dma.start()

    t = tids_ref[...]

    def shifted(x, fill, ks):
        sh_ref[:, 0:_PAD] = jnp.full((B, _PAD), fill, i32)
        sh_ref[:, _PAD:_PAD + S] = x
        sh_ref[:, _PAD + S:] = jnp.full((B, _PAD), fill, i32)
        return [sh_ref[:, _PAD + k:_PAD + k + S] for k in ks]

    def cummax(x):
        cm_ref[:, 0:S] = jnp.full((B, S), _NEG, i32)
        cur = x
        sh = 1
        while sh < S:
            cm_ref[:, S:] = cur
            cur = jnp.maximum(cur, cm_ref[:, S - sh:2 * S - sh])
            sh *= 2
        return cur

    pad = (t < 3) | (t >= 259)
    punct = ((t == ord('-') + _OFF) | (t == ord('.') + _OFF)
             | (t == ord(':') + _OFF) | (t == ord('=') + _OFF)
             | (t == ord(' ') + _OFF))
    bound = ((t == ord('{') + _OFF) | (t == ord('}') + _OFF)
             | (t == ord('(') + _OFF) | (t == ord(')') + _OFF)
             | (t == _OB) | (t == _CB))
    num = (t >= ord('0') + _OFF) & (t <= ord('9') + _OFF)
    base = jnp.where(pad, ROLE_PAD,
                     jnp.where(punct, ROLE_PUNCT,
                               jnp.where(bound, ROLE_BOUND,
                                         jnp.where(num, ROLE_NUM, ROLE_ALPHA))))

    iota = lax.broadcasted_iota(i32, (B, S), 1)

    tf = shifted(t, -1, list(range(0, 5)))
    m5 = (tf[0] == _GAP[0])
    for k in range(1, 5):
        m5 = m5 & (tf[k] == _GAP[k])
    mb = shifted(m5.astype(i32), 0, list(range(-4, 1)))
    dgap = (mb[4] | mb[3] | mb[2] | mb[1] | mb[0]) != 0

    is_dot = t == _DOT
    df = shifted(is_dot.astype(i32), 0, [1, 2])
    m3 = is_dot & (df[0] != 0) & (df[1] != 0)
    lnd = cummax(jnp.where(is_dot, -1, iota))
    sel3 = m3 & ((iota - lnd - 1) % 3 == 0)
    sb = shifted(sel3.astype(i32), 0, [-2, -1, 0])
    ddot = (sb[0] | sb[1] | sb[2]) != 0

    enc = jnp.where(t == _OB, 2 * iota + 1,
                    jnp.where(t == _CB, 2 * iota, -1))
    c = cummax(enc)
    markx = (t == _X) & (c >= 0) & ((c & 1) == 1)

    damage = dgap | ddot | markx
    role = jnp.where(damage, ROLE_DAMAGE, base)
    idx_ref[...] = role * V + t
    tbl_dma.wait()


def _build_prep(B, S, V, D, R):
    return pl.pallas_call(
        functools.partial(_prep_kernel, B, S, V, D, R),
        out_shape=(
            jax.ShapeDtypeStruct((R * V, D), jnp.float32),
            jax.ShapeDtypeStruct((B, S), jnp.int32),
        ),
        out_specs=(
            pl.BlockSpec(memory_space=pl.ANY),
            pl.BlockSpec(memory_space=pltpu.MemorySpace.VMEM),
        ),
        scratch_shapes=[
            pltpu.VMEM((B, 2 * S), jnp.int32),
            pltpu.VMEM((B, S + 2 * _PAD), jnp.int32),
            pltpu.VMEM((R * V, D), jnp.float32),
            pltpu.SemaphoreType.DMA,
        ],
    )


def _build_gather(N, D, table_rows):
    NC, NS = 2, 16
    NW = NC * NS
    assert N % NW == 0
    b_per_w = N // NW
    chunk = 16
    nbuf = 6
    assert b_per_w % chunk == 0
    n_chunks = b_per_w // chunk
    mesh = plsc.VectorSubcoreMesh(core_axis_name="c", subcore_axis_name="s")

    scratch = ([pltpu.VMEM((b_per_w,), jnp.int32)]
               + [pltpu.VMEM((chunk, D), jnp.float32)] * nbuf
               + [pltpu.SemaphoreType.DMA] * (2 * nbuf))

    @functools.partial(
        pl.kernel,
        out_type=jax.ShapeDtypeStruct((N, D), jnp.float32),
        mesh=mesh,
        scratch_types=scratch,
    )
    def gather(tbl_hbm, idx_hbm, out_hbm, idx_all, *bufs):
        rowb = bufs[0:nbuf]
        gsem = bufs[nbuf:2 * nbuf]
        ssem = bufs[2 * nbuf:3 * nbuf]
        wid = lax.axis_index("s") * NC + lax.axis_index("c")
        base = wid * b_per_w
        pltpu.sync_copy(idx_hbm.at[pl.ds(base, b_per_w)], idx_all)
        pend_g = [None] * nbuf
        pend_s = [None] * nbuf
        ahead = nbuf - 1
        for ci in range(min(ahead, n_chunks)):
            pend_g[ci % nbuf] = pltpu.async_copy(
                tbl_hbm.at[idx_all.at[pl.ds(ci * chunk, chunk)]],
                rowb[ci % nbuf], gsem[ci % nbuf])
        for ci in range(n_chunks):
            b = ci % nbuf
            pend_g[b].wait()
            pend_s[b] = pltpu.async_copy(rowb[b],
                                         out_hbm.at[pl.ds(base + ci * chunk, chunk)],
                                         ssem[b])
            nxt = ci + ahead
            if nxt < n_chunks:
                nb = nxt % nbuf
                if pend_s[nb] is not None:
                    pend_s[nb].wait()
                    pend_s[nb] = None
                pend_g[nb] = pltpu.async_copy(
                    tbl_hbm.at[idx_all.at[pl.ds(nxt * chunk, chunk)]],
                    rowb[nb], gsem[nb])
        for p in pend_s:
            if p is not None:
                p.wait()

    return gather


def kernel(input_ids, embed_table, role_table, gate_W, gate_b):
    B, S = input_ids.shape
    V, D = embed_table.shape
    R = role_table.shape[0]
    tids = input_ids.astype(jnp.int32)
    b2 = gate_b.reshape(1, D)
    tbl, fidx = _build_prep(B, S, V, D, R)(tids, embed_table, role_table, gate_W, b2)
    out = _build_gather(B * S, D, R * V)(tbl, fidx.reshape(B * S))
    return out.reshape(B, S, D)


# speedup vs baseline: 55.2456x; 1.0080x over previous
import functools

import jax
import jax.numpy as jnp
from jax import lax
from jax.experimental import pallas as pl
from jax.experimental.pallas import tpu as pltpu
from jax.experimental.pallas import tpu_sc as plsc

_OFF = 3
_DOT = ord('.') + _OFF
_OB = ord('[') + _OFF
_CB = ord(']') + _OFF
_X = ord('x') + _OFF
_GAP = tuple(ord(c) + _OFF for c in '<gap>')
_NEG = -(1 << 30)
_PAD = 128

ROLE_PAD, ROLE_ALPHA, ROLE_PUNCT, ROLE_BOUND, ROLE_DAMAGE, ROLE_NUM = 0, 1, 2, 3, 4, 5


def _prep_kernel(B, S, V, D, R,
                 tids_ref, e_ref, r_ref, w_ref, b_ref,
                 tbl_ref, idx_ref, cm_ref, sh_ref):
    t = tids_ref[...]
    i32 = jnp.int32

    def shifted(x, fill, ks):
        sh_ref[:, 0:_PAD] = jnp.full((B, _PAD), fill, i32)
        sh_ref[:, _PAD:_PAD + S] = x
        sh_ref[:, _PAD + S:] = jnp.full((B, _PAD), fill, i32)
        return [sh_ref[:, _PAD + k:_PAD + k + S] for k in ks]

    def cummax(x):
        cm_ref[:, 0:S] = jnp.full((B, S), _NEG, i32)
        cur = x
        sh = 1
        while sh < S:
            cm_ref[:, S:] = cur
            cur = jnp.maximum(cur, cm_ref[:, S - sh:2 * S - sh])
            sh *= 2
        return cur

    pad = (t < 3) | (t >= 259)
    punct = ((t == ord('-') + _OFF) | (t == ord('.') + _OFF)
             | (t == ord(':') + _OFF) | (t == ord('=') + _OFF)
             | (t == ord(' ') + _OFF))
    bound = ((t == ord('{') + _OFF) | (t == ord('}') + _OFF)
             | (t == ord('(') + _OFF) | (t == ord(')') + _OFF)
             | (t == _OB) | (t == _CB))
    num = (t >= ord('0') + _OFF) & (t <= ord('9') + _OFF)
    base = jnp.where(pad, ROLE_PAD,
                     jnp.where(punct, ROLE_PUNCT,
                               jnp.where(bound, ROLE_BOUND,
                                         jnp.where(num, ROLE_NUM, ROLE_ALPHA))))

    iota = lax.broadcasted_iota(i32, (B, S), 1)

    tf = shifted(t, -1, list(range(0, 5)))
    m5 = (tf[0] == _GAP[0])
    for k in range(1, 5):
        m5 = m5 & (tf[k] == _GAP[k])
    mb = shifted(m5.astype(i32), 0, list(range(-4, 1)))
    dgap = (mb[4] | mb[3] | mb[2] | mb[1] | mb[0]) != 0

    is_dot = t == _DOT
    df = shifted(is_dot.astype(i32), 0, [1, 2])
    m3 = is_dot & (df[0] != 0) & (df[1] != 0)
    lnd = cummax(jnp.where(is_dot, -1, iota))
    sel3 = m3 & ((iota - lnd - 1) % 3 == 0)
    sb = shifted(sel3.astype(i32), 0, [-2, -1, 0])
    ddot = (sb[0] | sb[1] | sb[2]) != 0

    enc = jnp.where(t == _OB, 2 * iota + 1,
                    jnp.where(t == _CB, 2 * iota, -1))
    c = cummax(enc)
    markx = (t == _X) & (c >= 0) & ((c & 1) == 1)

    damage = dgap | ddot | markx
    role = jnp.where(damage, ROLE_DAMAGE, base)
    idx_ref[...] = role * V + t

    e = e_ref[...]
    p = lax.dot_general(e, w_ref[:, 0:D], (((1,), (1,)), ((), ())),
                        preferred_element_type=jnp.float32)
    q = lax.dot_general(r_ref[...], w_ref[:, D:2 * D], (((1,), (1,)), ((), ())),
                        preferred_element_type=jnp.float32)
    bvec = b_ref[...]
    for r in range(R):
        gate = jax.nn.sigmoid(p + q[r:r + 1, :] + bvec)
        tbl_ref[r * V:(r + 1) * V, :] = e + gate * r_ref[r:r + 1, :]


def _build_prep(B, S, V, D, R):
    return pl.pallas_call(
        functools.partial(_prep_kernel, B, S, V, D, R),
        out_shape=(
            jax.ShapeDtypeStruct((R * V, D), jnp.float32),
            jax.ShapeDtypeStruct((B, S), jnp.int32),
        ),
        scratch_shapes=[
            pltpu.VMEM((B, 2 * S), jnp.int32),
            pltpu.VMEM((B, S + 2 * _PAD), jnp.int32),
        ],
    )


def _build_gather(N, D, table_rows):
    NC, NS = 2, 16
    NW = NC * NS
    assert N % NW == 0
    b_per_w = N // NW
    chunk = 16
    nbuf = 6
    assert b_per_w % chunk == 0
    n_chunks = b_per_w // chunk
    mesh = plsc.VectorSubcoreMesh(core_axis_name="c", subcore_axis_name="s")

    scratch = ([pltpu.VMEM((b_per_w,), jnp.int32)]
               + [pltpu.VMEM((chunk, D), jnp.float32)] * nbuf
               + [pltpu.SemaphoreType.DMA] * (2 * nbuf))

    @functools.partial(
        pl.kernel,
        out_type=jax.ShapeDtypeStruct((N, D), jnp.float32),
        mesh=mesh,
        scratch_types=scratch,
    )
    def gather(tbl_hbm, idx_hbm, out_hbm, idx_all, *bufs):
        rowb = bufs[0:nbuf]
        gsem = bufs[nbuf:2 * nbuf]
        ssem = bufs[2 * nbuf:3 * nbuf]
        wid = lax.axis_index("s") * NC + lax.axis_index("c")
        base = wid * b_per_w
        pltpu.sync_copy(idx_hbm.at[pl.ds(base, b_per_w)], idx_all)
        pend_g = [None] * nbuf
        pend_s = [None] * nbuf
        ahead = nbuf - 1
        for ci in range(min(ahead, n_chunks)):
            pend_g[ci % nbuf] = pltpu.async_copy(
                tbl_hbm.at[idx_all.at[pl.ds(ci * chunk, chunk)]],
                rowb[ci % nbuf], gsem[ci % nbuf])
        for ci in range(n_chunks):
            b = ci % nbuf
            pend_g[b].wait()
            pend_s[b] = pltpu.async_copy(rowb[b],
                                         out_hbm.at[pl.ds(base + ci * chunk, chunk)],
                                         ssem[b])
            nxt = ci + ahead
            if nxt < n_chunks:
                nb = nxt % nbuf
                if pend_s[nb] is not None:
                    pend_s[nb].wait()
                    pend_s[nb] = None
                pend_g[nb] = pltpu.async_copy(
                    tbl_hbm.at[idx_all.at[pl.ds(nxt * chunk, chunk)]],
                    rowb[nb], gsem[nb])
        for p in pend_s:
            if p is not None:
                p.wait()

    return gather


def kernel(input_ids, embed_table, role_table, gate_W, gate_b):
    B, S = input_ids.shape
    V, D = embed_table.shape
    R = role_table.shape[0]
    tids = input_ids.astype(jnp.int32)
    b2 = gate_b.reshape(1, D)
    tbl, fidx = _build_prep(B, S, V, D, R)(tids, embed_table, role_table, gate_W, b2)
    out = _build_gather(B * S, D, R * V)(tbl, fidx.reshape(B * S))
    return out.reshape(B, S, D)
